# Initial kernel scaffold; baseline (speedup 1.0000x reference)
import jax
import jax.numpy as jnp
from jax.experimental import pallas as pl


def kernel(prediction, truth):
    raise NotImplementedError("write your pallas kernel here")


# speedup vs baseline: 5.5089x; 5.5089x over previous
import functools

import jax
import jax.numpy as jnp
from jax import lax
from jax.experimental import pallas as pl
from jax.experimental.pallas import tpu as pltpu
from jax.experimental.pallas import tpu_sc as plsc

N_BINS = 10
B_BLK = 16
NC, NS, L = 2, 16, 16


def _membership_body(hw_ref, pred_ref, truth_ref, m_ref):
    p = pred_ref[...]
    t = truth_ref[...]
    lt = p < t[:, :, None]
    c = jnp.sum(lt.astype(jnp.int32), axis=2)
    lo = jnp.max(jnp.where(lt, p, -jnp.inf), axis=2)
    hi = jnp.min(jnp.where(lt, jnp.inf, p), axis=2)
    k = c // 10
    is_b = (c == k * 10) & (k >= 1) & (k <= 9)
    kc = jnp.clip(k, 1, 9)
    hwk = jnp.zeros_like(t)
    for j in range(1, N_BINS):
        hwk = jnp.where(kc == j, hw_ref[0, j], hwk)
    lwk = 1.0 - hwk
    q = lo * lwk + hi * hwk
    m_boundary = k + (t > q).astype(jnp.int32)
    m_base = 1 + jnp.maximum(c - 1, 0) // 10
    m_ref[...] = jnp.where(is_b, m_boundary, m_base)


def _sc_bincount_body(m_hbm, out_hbm, chunk_v, res_v):
    cid = lax.axis_index("c")
    sid = lax.axis_index("s")
    wid = cid * NS + sid
    ch = chunk_v.shape[0]
    pltpu.sync_copy(m_hbm.at[pl.ds(wid * ch, ch)], chunk_v)

    ones = jnp.full((L,), 1, jnp.int32)
    zeros = jnp.full((L,), 0, jnp.int32)

    def body(i, accs):
        v = chunk_v[pl.ds(i * L, L)]
        return tuple(acc + jnp.where(v >= jnp.full((L,), b + 1, jnp.int32),
                                     ones, zeros)
                     for b, acc in enumerate(accs))

    accs0 = tuple(jnp.zeros((L,), jnp.int32) for _ in range(N_BINS))
    accs = lax.fori_loop(0, ch // L, body, accs0)

    lanes = lax.iota(jnp.int32, L)
    perms = [lanes ^ jnp.full((L,), p, jnp.int32) for p in (1, 2, 4, 8)]

    dnums = lax.GatherDimensionNumbers(
        offset_dims=(), collapsed_slice_dims=(0,), start_index_map=(0,))

    def lanesum(v):
        for p in perms:
            v = v + lax.gather(v, p[:, None], dnums, slice_sizes=(1,),
                               mode=lax.GatherScatterMode.PROMISE_IN_BOUNDS)
        return v

    totals = [lanesum(a) for a in accs] + [zeros]
    res = zeros
    for b in range(N_BINS):
        res = jnp.where(lanes == jnp.full((L,), b, jnp.int32),
                        totals[b] - totals[b + 1], res)
    res_v[...] = res
    pltpu.sync_copy(res_v, out_hbm.at[wid])


def kernel(prediction, truth):
    B, D, S = prediction.shape

    qs = jnp.linspace(0.0, 1.0, N_BINS + 1)
    pos = qs * (S - 1)
    hw = pos - jnp.floor(pos)
    hw16 = jnp.zeros((1, 16), jnp.float32).at[0, : N_BINS + 1].set(hw)

    membership = pl.pallas_call(
        _membership_body,
        grid=(B // B_BLK,),
        in_specs=[
            pl.BlockSpec(memory_space=pltpu.SMEM),
            pl.BlockSpec((B_BLK, D, S), lambda i: (i, 0, 0)),
            pl.BlockSpec((B_BLK, D), lambda i: (i, 0)),
        ],
        out_specs=pl.BlockSpec((B_BLK, D), lambda i: (i, 0)),
        out_shape=jax.ShapeDtypeStruct((B, D), jnp.int32),
    )(hw16, prediction, truth)

    tot = B * D
    ch = tot // (NC * NS)
    mesh = plsc.VectorSubcoreMesh(core_axis_name="c", subcore_axis_name="s")
    sc_bincount = functools.partial(
        pl.kernel,
        mesh=mesh,
        out_type=jax.ShapeDtypeStruct((NC * NS, L), jnp.int32),
        scratch_types=[
            pltpu.VMEM((ch,), jnp.int32),
            pltpu.VMEM((L,), jnp.int32),
        ],
    )(_sc_bincount_body)
    partial_counts = sc_bincount(membership.reshape(tot))

    return partial_counts.sum(axis=0)[:N_BINS].astype(jnp.float32)
